# Initial kernel scaffold; baseline (speedup 1.0000x reference)
import jax
import jax.numpy as jnp
from jax.experimental import pallas as pl


def kernel(x, edge_index0, edge_weight0, edge_index1, edge_weight1, W):
    raise NotImplementedError("write your pallas kernel here")


# speedup vs baseline: 3.7806x; 3.7806x over previous
import functools

import jax
import jax.numpy as jnp
from jax import lax
from jax.experimental import pallas as pl
from jax.experimental.pallas import tpu as pltpu
from jax.experimental.pallas import tpu_sc as plsc

N = 10000
D = 128
F = 64
NCLS = 2
E = 160000
NS = 16
CH = 128
EPT = 10240
CHUNKS = EPT // CH
EP = EPT * NS
NPAD = 10240
RPT = NPAD // NS


def _matmul(x, W):
    def body(x_ref, w_ref, o_ref):
        o_ref[...] = jnp.dot(x_ref[...], w_ref[0],
                             preferred_element_type=jnp.float32)

    Ws = jnp.stack([W[:, :F], W[:, F:]])
    return pl.pallas_call(
        body,
        grid=(NCLS,),
        in_specs=[
            pl.BlockSpec((N, D), lambda c: (0, 0)),
            pl.BlockSpec((1, D, F), lambda c: (c, 0, 0)),
        ],
        out_specs=pl.BlockSpec((N, F), lambda c: (c, 0)),
        out_shape=jax.ShapeDtypeStruct((NCLS * N, F), jnp.float32),
    )(x, Ws)


def _prep_edges(ei, ew, c):
    pad = EP - E
    cols = jnp.pad(ei[1], (0, pad)) + c * N
    rows = jnp.pad(ei[0], (0, pad))
    w = jnp.pad(ew, (0, pad))
    return (cols.reshape(NS, CHUNKS, CH),
            rows.reshape(NS, CHUNKS, CH),
            w.reshape(NS, CHUNKS, CH))


def _sc_spmm(tmp, cols, rows, wts, zeros):
    mesh = plsc.VectorSubcoreMesh(core_axis_name="c", subcore_axis_name="s")

    @functools.partial(
        pl.kernel,
        out_type=jax.ShapeDtypeStruct((NCLS, NPAD, F), jnp.float32),
        mesh=mesh,
        scratch_types=[
            pltpu.VMEM((CHUNKS, CH), jnp.int32),
            pltpu.VMEM((CHUNKS, CH), jnp.int32),
            pltpu.VMEM((CHUNKS, CH), jnp.float32),
            pltpu.VMEM((CH, F), jnp.float32),
            pltpu.VMEM_SHARED((NPAD, F), jnp.float32),
            pltpu.SemaphoreType.DMA,
        ],
        compiler_params=pltpu.CompilerParams(use_tc_tiling_on_sc=False),
    )
    def k(tmp_hbm, cols_hbm, rows_hbm, w_hbm, zeros_hbm, out_hbm,
          cols_v, rows_v, w_v, buf_v, acc_sh, sem):
        c = lax.axis_index("c")
        s = lax.axis_index("s")
        pltpu.sync_copy(cols_hbm.at[c, s], cols_v)
        pltpu.sync_copy(rows_hbm.at[c, s], rows_v)
        pltpu.sync_copy(w_hbm.at[c, s], w_v)
        pltpu.sync_copy(zeros_hbm.at[pl.ds(s * RPT, RPT)],
                        acc_sh.at[pl.ds(s * RPT, RPT)])
        plsc.subcore_barrier()

        def chunk_body(j, _):
            pltpu.async_copy(tmp_hbm.at[cols_v.at[j]], buf_v, sem).wait()

            def mul(g, _):
                wv = w_v[j, pl.ds(g * 16, 16)]
                for e16 in range(16):
                    w = jnp.broadcast_to(wv[e16], (16,))
                    e = g * 16 + e16
                    for f in range(F // 16):
                        sl = pl.ds(f * 16, 16)
                        buf_v[e, sl] = buf_v[e, sl] * w
                return ()

            lax.fori_loop(0, CH // 16, mul, ())
            pltpu.sync_copy(buf_v, acc_sh.at[rows_v.at[j]], add=True)
            return ()

        lax.fori_loop(0, CHUNKS, chunk_body, ())
        plsc.subcore_barrier()
        pltpu.sync_copy(acc_sh.at[pl.ds(s * RPT, RPT)],
                        out_hbm.at[c, pl.ds(s * RPT, RPT)])

    return k(tmp, cols, rows, wts, zeros)


@jax.jit
def kernel(x, edge_index0, edge_weight0, edge_index1, edge_weight1, W):
    tmp = _matmul(x, W)
    c0, r0, w0 = _prep_edges(edge_index0, edge_weight0, 0)
    c1, r1, w1 = _prep_edges(edge_index1, edge_weight1, 1)
    cols = jnp.stack([c0, c1])
    rows = jnp.stack([r0, r1])
    wts = jnp.stack([w0, w1])
    zeros = jnp.zeros((NPAD, F), jnp.float32)
    out = _sc_spmm(tmp, cols, rows, wts, zeros)
    return jnp.transpose(out[:, :N, :], (1, 0, 2)).reshape(N, NCLS * F)


# speedup vs baseline: 5.2570x; 1.3905x over previous
import functools

import jax
import jax.numpy as jnp
from jax import lax
from jax.experimental import pallas as pl
from jax.experimental.pallas import tpu as pltpu
from jax.experimental.pallas import tpu_sc as plsc

N = 10000
D = 128
F = 64
NCLS = 2
E = 160000
NS = 16
CH = 128
EPT = 10240
CHUNKS = EPT // CH
EP = EPT * NS
NPAD = 10240
RPT = NPAD // NS
NBUF = 2


def _matmul(x, W):
    def body(x_ref, w_ref, o_ref):
        o_ref[...] = jnp.dot(x_ref[...], w_ref[0],
                             preferred_element_type=jnp.float32)

    Ws = jnp.stack([W[:, :F], W[:, F:]])
    return pl.pallas_call(
        body,
        grid=(NCLS,),
        in_specs=[
            pl.BlockSpec((N, D), lambda c: (0, 0)),
            pl.BlockSpec((1, D, F), lambda c: (c, 0, 0)),
        ],
        out_specs=pl.BlockSpec((N, F), lambda c: (c, 0)),
        out_shape=jax.ShapeDtypeStruct((NCLS * N, F), jnp.float32),
    )(x, Ws)


def _prep_edges(ei, ew, c):
    pad = EP - E
    cols = jnp.pad(ei[1], (0, pad)) + c * N
    rows = jnp.pad(ei[0], (0, pad))
    w = jnp.pad(ew, (0, pad))
    return (cols.reshape(NS, CHUNKS, CH),
            rows.reshape(NS, CHUNKS, CH),
            w.reshape(NS, CHUNKS, CH))


def _sc_spmm(tmp, cols, rows, wts, zeros):
    mesh = plsc.VectorSubcoreMesh(core_axis_name="c", subcore_axis_name="s")

    @functools.partial(
        pl.kernel,
        out_type=jax.ShapeDtypeStruct((NCLS, NPAD, F), jnp.float32),
        mesh=mesh,
        scratch_types=[
            pltpu.VMEM((CHUNKS, CH), jnp.int32),
            pltpu.VMEM((CHUNKS, CH), jnp.int32),
            pltpu.VMEM((CHUNKS, CH), jnp.float32),
            pltpu.VMEM((NBUF, CH, F), jnp.float32),
            pltpu.VMEM_SHARED((NPAD, F), jnp.float32),
            pltpu.SemaphoreType.DMA,
            pltpu.SemaphoreType.DMA,
        ],
        compiler_params=pltpu.CompilerParams(use_tc_tiling_on_sc=False),
    )
    def k(tmp_hbm, cols_hbm, rows_hbm, w_hbm, zeros_hbm, out_hbm,
          cols_v, rows_v, w_v, buf_v, acc_sh, sem0, sem1):
        c = lax.axis_index("c")
        s = lax.axis_index("s")
        sems = (sem0, sem1)
        pltpu.sync_copy(cols_hbm.at[c, s], cols_v)
        pltpu.sync_copy(rows_hbm.at[c, s], rows_v)
        pltpu.sync_copy(w_hbm.at[c, s], w_v)
        pltpu.sync_copy(zeros_hbm.at[pl.ds(s * RPT, RPT)],
                        acc_sh.at[pl.ds(s * RPT, RPT)])
        plsc.subcore_barrier()

        def compute(j, b):
            def mul(g, _):
                wv = w_v[j, pl.ds(g * 16, 16)]
                for e16 in range(16):
                    w = jnp.broadcast_to(wv[e16], (16,))
                    e = g * 16 + e16
                    for f in range(F // 16):
                        sl = pl.ds(f * 16, 16)
                        buf_v[b, e, sl] = buf_v[b, e, sl] * w
                return ()

            lax.fori_loop(0, CH // 16, mul, ())
            pltpu.sync_copy(buf_v.at[b], acc_sh.at[rows_v.at[j]], add=True)

        def start(j, b):
            pltpu.async_copy(tmp_hbm.at[cols_v.at[j]], buf_v.at[b], sems[b])

        def wait(b):
            pltpu.make_async_copy(
                tmp_hbm.at[cols_v.at[0]], buf_v.at[b], sems[b]).wait()

        for b in range(NBUF):
            start(b, b)

        def ring_body(i, _):
            j = i * NBUF
            for b in range(NBUF):
                wait(b)
                compute(j + b, b)
                start(j + b + NBUF, b)
            return ()

        lax.fori_loop(0, CHUNKS // NBUF - 1, ring_body, ())
        for b in range(NBUF):
            wait(b)
            compute(CHUNKS - NBUF + b, b)
        plsc.subcore_barrier()
        pltpu.sync_copy(acc_sh.at[pl.ds(s * RPT, RPT)],
                        out_hbm.at[c, pl.ds(s * RPT, RPT)])

    return k(tmp, cols, rows, wts, zeros)


@jax.jit
def kernel(x, edge_index0, edge_weight0, edge_index1, edge_weight1, W):
    tmp = _matmul(x, W)
    c0, r0, w0 = _prep_edges(edge_index0, edge_weight0, 0)
    c1, r1, w1 = _prep_edges(edge_index1, edge_weight1, 1)
    cols = jnp.stack([c0, c1])
    rows = jnp.stack([r0, r1])
    wts = jnp.stack([w0, w1])
    zeros = jnp.zeros((NPAD, F), jnp.float32)
    out = _sc_spmm(tmp, cols, rows, wts, zeros)
    return jnp.transpose(out[:, :N, :], (1, 0, 2)).reshape(N, NCLS * F)


# speedup vs baseline: 6.0995x; 1.1603x over previous
import functools

import jax
import jax.numpy as jnp
from jax import lax
from jax.experimental import pallas as pl
from jax.experimental.pallas import tpu as pltpu
from jax.experimental.pallas import tpu_sc as plsc

N = 10000
D = 128
F = 64
NCLS = 2
E = 160000
NS = 16
CH = 128
EPT = 10240
CHUNKS = EPT // CH
EP = EPT * NS
NPAD = 10240
RPT = NPAD // NS
NBUF = 4


def _matmul(x, W):
    def body(x_ref, w_ref, o_ref):
        o_ref[...] = jnp.dot(x_ref[...], w_ref[0],
                             preferred_element_type=jnp.float32)

    Ws = jnp.stack([W[:, :F], W[:, F:]])
    return pl.pallas_call(
        body,
        grid=(NCLS,),
        in_specs=[
            pl.BlockSpec((N, D), lambda c: (0, 0)),
            pl.BlockSpec((1, D, F), lambda c: (c, 0, 0)),
        ],
        out_specs=pl.BlockSpec((N, F), lambda c: (c, 0)),
        out_shape=jax.ShapeDtypeStruct((NCLS * N, F), jnp.float32),
    )(x, Ws)


def _prep_edges(ei, ew, c):
    pad = EP - E
    cols = jnp.pad(ei[1], (0, pad)) + c * N
    rows = jnp.pad(ei[0], (0, pad))
    w = jnp.pad(ew, (0, pad))
    return (cols.reshape(NS, CHUNKS, CH),
            rows.reshape(NS, CHUNKS, CH),
            w.reshape(NS, CHUNKS, CH))


def _sc_spmm(tmp, cols, rows, wts, zeros):
    mesh = plsc.VectorSubcoreMesh(core_axis_name="c", subcore_axis_name="s")

    @functools.partial(
        pl.kernel,
        out_type=jax.ShapeDtypeStruct((NCLS, NPAD, F), jnp.float32),
        mesh=mesh,
        scratch_types=[
            pltpu.VMEM((CHUNKS, CH), jnp.int32),
            pltpu.VMEM((CHUNKS, CH), jnp.int32),
            pltpu.VMEM((CHUNKS, CH), jnp.float32),
            pltpu.VMEM((NBUF, CH, F), jnp.float32),
            pltpu.VMEM_SHARED((NPAD, F), jnp.float32),
            pltpu.SemaphoreType.DMA,
            pltpu.SemaphoreType.DMA,
            pltpu.SemaphoreType.DMA,
            pltpu.SemaphoreType.DMA,
        ],
        compiler_params=pltpu.CompilerParams(use_tc_tiling_on_sc=False),
    )
    def k(tmp_hbm, cols_hbm, rows_hbm, w_hbm, zeros_hbm, out_hbm,
          cols_v, rows_v, w_v, buf_v, acc_sh, sem0, sem1, sem2, sem3):
        c = lax.axis_index("c")
        s = lax.axis_index("s")
        sems = (sem0, sem1, sem2, sem3)
        pltpu.sync_copy(cols_hbm.at[c, s], cols_v)
        pltpu.sync_copy(rows_hbm.at[c, s], rows_v)
        pltpu.sync_copy(w_hbm.at[c, s], w_v)
        pltpu.sync_copy(zeros_hbm.at[pl.ds(s * RPT, RPT)],
                        acc_sh.at[pl.ds(s * RPT, RPT)])
        plsc.subcore_barrier()

        def compute(j, b):
            def mul(g, _):
                wv = w_v[j, pl.ds(g * 16, 16)]
                for e16 in range(16):
                    w = jnp.broadcast_to(wv[e16], (16,))
                    e = g * 16 + e16
                    for f in range(F // 16):
                        sl = pl.ds(f * 16, 16)
                        buf_v[b, e, sl] = buf_v[b, e, sl] * w
                return ()

            lax.fori_loop(0, CH // 16, mul, ())
            pltpu.sync_copy(buf_v.at[b], acc_sh.at[rows_v.at[j]], add=True)

        def start(j, b):
            pltpu.async_copy(tmp_hbm.at[cols_v.at[j]], buf_v.at[b], sems[b])

        def wait(b):
            pltpu.make_async_copy(
                tmp_hbm.at[cols_v.at[0]], buf_v.at[b], sems[b]).wait()

        for b in range(NBUF):
            start(b, b)

        def ring_body(i, _):
            j = i * NBUF
            for b in range(NBUF):
                wait(b)
                compute(j + b, b)
                start(j + b + NBUF, b)
            return ()

        lax.fori_loop(0, CHUNKS // NBUF - 1, ring_body, ())
        for b in range(NBUF):
            wait(b)
            compute(CHUNKS - NBUF + b, b)
        plsc.subcore_barrier()
        pltpu.sync_copy(acc_sh.at[pl.ds(s * RPT, RPT)],
                        out_hbm.at[c, pl.ds(s * RPT, RPT)])

    return k(tmp, cols, rows, wts, zeros)


@jax.jit
def kernel(x, edge_index0, edge_weight0, edge_index1, edge_weight1, W):
    tmp = _matmul(x, W)
    c0, r0, w0 = _prep_edges(edge_index0, edge_weight0, 0)
    c1, r1, w1 = _prep_edges(edge_index1, edge_weight1, 1)
    cols = jnp.stack([c0, c1])
    rows = jnp.stack([r0, r1])
    wts = jnp.stack([w0, w1])
    zeros = jnp.zeros((NPAD, F), jnp.float32)
    out = _sc_spmm(tmp, cols, rows, wts, zeros)
    return jnp.transpose(out[:, :N, :], (1, 0, 2)).reshape(N, NCLS * F)


# speedup vs baseline: 6.4505x; 1.0575x over previous
import functools

import jax
import jax.numpy as jnp
from jax import lax
from jax.experimental import pallas as pl
from jax.experimental.pallas import tpu as pltpu
from jax.experimental.pallas import tpu_sc as plsc

N = 10000
D = 128
F = 64
NCLS = 2
E = 160000
NS = 16
CH = 128
EPT = 10240
CHUNKS = EPT // CH
EP = EPT * NS
NPAD = 10240
RPT = NPAD // NS
NBUF = 5


def _matmul(x, W):
    def body(x_ref, w_ref, o_ref):
        o_ref[...] = jnp.dot(x_ref[...], w_ref[0],
                             preferred_element_type=jnp.float32)

    Ws = jnp.stack([W[:, :F], W[:, F:]])
    return pl.pallas_call(
        body,
        grid=(NCLS,),
        in_specs=[
            pl.BlockSpec((N, D), lambda c: (0, 0)),
            pl.BlockSpec((1, D, F), lambda c: (c, 0, 0)),
        ],
        out_specs=pl.BlockSpec((N, F), lambda c: (c, 0)),
        out_shape=jax.ShapeDtypeStruct((NCLS * N, F), jnp.float32),
    )(x, Ws)


def _prep_edges(ei, ew, c):
    pad = EP - E
    cols = jnp.pad(ei[1], (0, pad)) + c * N
    rows = jnp.pad(ei[0], (0, pad))
    w = jnp.pad(ew, (0, pad))
    return (cols.reshape(NS, CHUNKS, CH),
            rows.reshape(NS, CHUNKS, CH),
            w.reshape(NS, CHUNKS, CH))


def _sc_spmm(tmp, cols, rows, wts, zeros):
    mesh = plsc.VectorSubcoreMesh(core_axis_name="c", subcore_axis_name="s")

    @functools.partial(
        pl.kernel,
        out_type=jax.ShapeDtypeStruct((NCLS, NPAD, F), jnp.float32),
        mesh=mesh,
        scratch_types=[
            pltpu.VMEM((CHUNKS, CH), jnp.int32),
            pltpu.VMEM((CHUNKS, CH), jnp.int32),
            pltpu.VMEM((CHUNKS, CH), jnp.float32),
            pltpu.VMEM((NBUF, CH, F), jnp.float32),
            pltpu.VMEM_SHARED((NPAD, F), jnp.float32),
            pltpu.SemaphoreType.DMA,
            pltpu.SemaphoreType.DMA,
            pltpu.SemaphoreType.DMA,
            pltpu.SemaphoreType.DMA,
            pltpu.SemaphoreType.DMA,
            pltpu.SemaphoreType.DMA,
            pltpu.SemaphoreType.DMA,
            pltpu.SemaphoreType.DMA,
        ],
        compiler_params=pltpu.CompilerParams(use_tc_tiling_on_sc=False),
    )
    def k(tmp_hbm, cols_hbm, rows_hbm, w_hbm, zeros_hbm, out_hbm,
          cols_v, rows_v, w_v, buf_v, acc_sh, *sems):
        c = lax.axis_index("c")
        s = lax.axis_index("s")
        pltpu.sync_copy(cols_hbm.at[c, s], cols_v)
        pltpu.sync_copy(rows_hbm.at[c, s], rows_v)
        pltpu.sync_copy(w_hbm.at[c, s], w_v)
        pltpu.sync_copy(zeros_hbm.at[pl.ds(s * RPT, RPT)],
                        acc_sh.at[pl.ds(s * RPT, RPT)])
        plsc.subcore_barrier()

        def compute(j, b):
            def mul(g, _):
                wv = w_v[j, pl.ds(g * 16, 16)]
                for e16 in range(16):
                    w = jnp.broadcast_to(wv[e16], (16,))
                    e = g * 16 + e16
                    for f in range(F // 16):
                        sl = pl.ds(f * 16, 16)
                        buf_v[b, e, sl] = buf_v[b, e, sl] * w
                return ()

            lax.fori_loop(0, CH // 16, mul, ())
            pltpu.sync_copy(buf_v.at[b], acc_sh.at[rows_v.at[j]], add=True)

        def start(j, b):
            pltpu.async_copy(tmp_hbm.at[cols_v.at[j]], buf_v.at[b], sems[b])

        def wait(b):
            pltpu.make_async_copy(
                tmp_hbm.at[cols_v.at[0]], buf_v.at[b], sems[b]).wait()

        for b in range(NBUF):
            start(b, b)

        def ring_body(i, _):
            j = i * NBUF
            for b in range(NBUF):
                wait(b)
                compute(j + b, b)
                start(j + b + NBUF, b)
            return ()

        lax.fori_loop(0, CHUNKS // NBUF - 1, ring_body, ())
        for b in range(NBUF):
            wait(b)
            compute(CHUNKS - NBUF + b, b)
        plsc.subcore_barrier()
        pltpu.sync_copy(acc_sh.at[pl.ds(s * RPT, RPT)],
                        out_hbm.at[c, pl.ds(s * RPT, RPT)])

    return k(tmp, cols, rows, wts, zeros)


@jax.jit
def kernel(x, edge_index0, edge_weight0, edge_index1, edge_weight1, W):
    tmp = _matmul(x, W)
    c0, r0, w0 = _prep_edges(edge_index0, edge_weight0, 0)
    c1, r1, w1 = _prep_edges(edge_index1, edge_weight1, 1)
    cols = jnp.stack([c0, c1])
    rows = jnp.stack([r0, r1])
    wts = jnp.stack([w0, w1])
    zeros = jnp.zeros((NPAD, F), jnp.float32)
    out = _sc_spmm(tmp, cols, rows, wts, zeros)
    return jnp.transpose(out[:, :N, :], (1, 0, 2)).reshape(N, NCLS * F)


# speedup vs baseline: 6.5539x; 1.0160x over previous
import functools

import jax
import jax.numpy as jnp
from jax import lax
from jax.experimental import pallas as pl
from jax.experimental.pallas import tpu as pltpu
from jax.experimental.pallas import tpu_sc as plsc

N = 10000
D = 128
F = 64
NCLS = 2
E = 160000
NS = 16
CH = 128
EPT = 10240
CHUNKS = EPT // CH
EP = EPT * NS
NPAD = 10240
RPT = NPAD // NS
K = 5
L = 4


def _matmul(x, W):
    def body(x_ref, w_ref, o_ref):
        o_ref[...] = jnp.dot(x_ref[...], w_ref[0],
                             preferred_element_type=jnp.float32)

    Ws = jnp.stack([W[:, :F], W[:, F:]])
    return pl.pallas_call(
        body,
        grid=(NCLS,),
        in_specs=[
            pl.BlockSpec((N, D), lambda c: (0, 0)),
            pl.BlockSpec((1, D, F), lambda c: (c, 0, 0)),
        ],
        out_specs=pl.BlockSpec((N, F), lambda c: (c, 0)),
        out_shape=jax.ShapeDtypeStruct((NCLS * N, F), jnp.float32),
    )(x, Ws)


def _prep_edges(ei, ew, c):
    pad = EP - E
    cols = jnp.pad(ei[1], (0, pad)) + c * N
    rows = jnp.pad(ei[0], (0, pad))
    w = jnp.pad(ew, (0, pad))
    return (cols.reshape(NS, CHUNKS, CH),
            rows.reshape(NS, CHUNKS, CH),
            w.reshape(NS, CHUNKS, CH))


def _sc_spmm(tmp, cols, rows, wts, zeros):
    mesh = plsc.VectorSubcoreMesh(core_axis_name="c", subcore_axis_name="s")

    @functools.partial(
        pl.kernel,
        out_type=jax.ShapeDtypeStruct((NCLS, NPAD, F), jnp.float32),
        mesh=mesh,
        scratch_types=[
            pltpu.VMEM((CHUNKS, CH), jnp.int32),
            pltpu.VMEM((CHUNKS, CH), jnp.int32),
            pltpu.VMEM((CHUNKS, CH), jnp.float32),
            pltpu.VMEM((K, CH, F), jnp.float32),
            pltpu.VMEM_SHARED((NPAD, F), jnp.float32),
        ] + [pltpu.SemaphoreType.DMA] * (2 * K),
        compiler_params=pltpu.CompilerParams(use_tc_tiling_on_sc=False),
    )
    def k(tmp_hbm, cols_hbm, rows_hbm, w_hbm, zeros_hbm, out_hbm,
          cols_v, rows_v, w_v, buf_v, acc_sh, *sems):
        gsems = sems[:K]
        ssems = sems[K:]
        c = lax.axis_index("c")
        s = lax.axis_index("s")
        pltpu.sync_copy(cols_hbm.at[c, s], cols_v)
        pltpu.sync_copy(rows_hbm.at[c, s], rows_v)
        pltpu.sync_copy(w_hbm.at[c, s], w_v)
        pltpu.sync_copy(zeros_hbm.at[pl.ds(s * RPT, RPT)],
                        acc_sh.at[pl.ds(s * RPT, RPT)])
        plsc.subcore_barrier()

        def mul(j, b):
            def grp(g, _):
                wv = w_v[j, pl.ds(g * 16, 16)]
                for e16 in range(16):
                    w = jnp.broadcast_to(wv[e16], (16,))
                    e = g * 16 + e16
                    for f in range(F // 16):
                        sl = pl.ds(f * 16, 16)
                        buf_v[b, e, sl] = buf_v[b, e, sl] * w
                return ()

            lax.fori_loop(0, CH // 16, grp, ())

        def start_gather(j, b):
            pltpu.async_copy(tmp_hbm.at[cols_v.at[j]], buf_v.at[b], gsems[b])

        def wait_gather(b):
            pltpu.make_async_copy(
                tmp_hbm.at[cols_v.at[0]], buf_v.at[b], gsems[b]).wait()

        def start_scatter(j, b):
            pltpu.async_copy(buf_v.at[b], acc_sh.at[rows_v.at[j]], ssems[b],
                             add=True)

        def wait_scatter(b):
            pltpu.make_async_copy(
                buf_v.at[b], acc_sh.at[rows_v.at[0]], ssems[b]).wait()

        def step(j, b, wait_sc, issue_g):
            wait_gather(b)
            mul(j, b)
            start_scatter(j, b)
            bg = (b + L) % K
            if wait_sc:
                wait_scatter(bg)
            if issue_g:
                start_gather(j + L, bg)

        for j in range(L):
            start_gather(j, j % K)

        for b in range(K):
            step(b, b, wait_sc=(b >= K - L), issue_g=True)

        def ring_body(i, _):
            j0 = i * K
            for b in range(K):
                step(j0 + b, b, wait_sc=True, issue_g=True)
            return ()

        lax.fori_loop(1, CHUNKS // K - 1, ring_body, ())

        for b in range(K):
            step(CHUNKS - K + b, b, wait_sc=(b < K - L), issue_g=(b < K - L))
        for b in range(K):
            wait_scatter(b)
        plsc.subcore_barrier()
        pltpu.sync_copy(acc_sh.at[pl.ds(s * RPT, RPT)],
                        out_hbm.at[c, pl.ds(s * RPT, RPT)])

    return k(tmp, cols, rows, wts, zeros)


@jax.jit
def kernel(x, edge_index0, edge_weight0, edge_index1, edge_weight1, W):
    tmp = _matmul(x, W)
    c0, r0, w0 = _prep_edges(edge_index0, edge_weight0, 0)
    c1, r1, w1 = _prep_edges(edge_index1, edge_weight1, 1)
    cols = jnp.stack([c0, c1])
    rows = jnp.stack([r0, r1])
    wts = jnp.stack([w0, w1])
    zeros = jnp.zeros((NPAD, F), jnp.float32)
    out = _sc_spmm(tmp, cols, rows, wts, zeros)
    return jnp.transpose(out[:, :N, :], (1, 0, 2)).reshape(N, NCLS * F)
